# baseline (speedup 1.0000x reference)
import functools

import jax
import jax.numpy as jnp
from jax import lax
from jax.experimental import pallas as pl
from jax.experimental.pallas import tpu as pltpu
from jax.experimental.pallas import tpu_sc as plsc

N = 10000
E = 320000
D = 128
C = 64
B = 4096

NCORE = 2
NSUB = 16
NW = NCORE * NSUB
CHW = 80
NCH = E // (NW * CHW)
MW = 128
RPS = 640
NPAD = NSUB * RPS
BW = B // NW
PBW = 128
FCAP = NCH * CHW + 2 * PBW
SB = 25

_mesh = plsc.VectorSubcoreMesh(core_axis_name="c", subcore_axis_name="s")
_sc_params = pltpu.CompilerParams(
    use_tc_tiling_on_sc=False, needs_layout_passes=False)


@functools.partial(
    pl.kernel,
    mesh=_mesh,
    out_type=[
        jax.ShapeDtypeStruct((NPAD,), jnp.float32),
        jax.ShapeDtypeStruct((NPAD,), jnp.float32),
        jax.ShapeDtypeStruct((NPAD,), jnp.float32),
    ],
    scratch_types=[
        pltpu.VMEM((NCH, CHW), jnp.int32),
        pltpu.VMEM((NCH, CHW), jnp.float32),
        pltpu.VMEM((B // NSUB // MW, MW), jnp.int32),
        pltpu.VMEM((RPS,), jnp.float32),
        pltpu.VMEM((MW,), jnp.float32),
        pltpu.VMEM_SHARED((NPAD,), jnp.float32),
        pltpu.VMEM_SHARED((NPAD,), jnp.float32),
    ],
    compiler_params=_sc_params,
)
def _deg(col2d, ew2d, nodes2d, degp0, degp1, mark, col_v, ew_v, node_v,
         zero_v, ones_v, deg_s, mark_s):
    c = lax.axis_index("c")
    s = lax.axis_index("s")
    wid = c * NSUB + s
    nrows = B // NSUB // MW
    pltpu.sync_copy(col2d.at[pl.ds(wid * NCH, NCH)], col_v)
    pltpu.sync_copy(ew2d.at[pl.ds(wid * NCH, NCH)], ew_v)
    pltpu.sync_copy(nodes2d.at[pl.ds(s * nrows, nrows)], node_v)

    def zb(i, carry):
        zero_v[pl.ds(i * 16, 16)] = jnp.zeros((16,), jnp.float32)
        return carry

    lax.fori_loop(0, RPS // 16, zb, None)
    for i in range(MW // 16):
        ones_v[pl.ds(i * 16, 16)] = jnp.full((16,), 1.0, jnp.float32)
    pltpu.sync_copy(zero_v, deg_s.at[pl.ds(s * RPS, RPS)])
    pltpu.sync_copy(zero_v, mark_s.at[pl.ds(s * RPS, RPS)])
    plsc.subcore_barrier()

    def chunk(j, carry):
        pltpu.sync_copy(ew_v.at[j], deg_s.at[col_v.at[j]], add=True)
        return carry

    lax.fori_loop(0, NCH, chunk, None)
    for k in range(B // NSUB // MW):
        pltpu.sync_copy(ones_v, mark_s.at[node_v.at[k]], add=True)
    plsc.subcore_barrier()

    @pl.when(c == 0)
    def _():
        pltpu.sync_copy(deg_s.at[pl.ds(s * RPS, RPS)], degp0.at[pl.ds(s * RPS, RPS)])
        pltpu.sync_copy(mark_s.at[pl.ds(s * RPS, RPS)], mark.at[pl.ds(s * RPS, RPS)])

    @pl.when(c == 1)
    def _():
        pltpu.sync_copy(deg_s.at[pl.ds(s * RPS, RPS)], degp1.at[pl.ds(s * RPS, RPS)])


def _prep_body(emb_ref, w_ref, degp0_ref, degp1_ref, mark_ref,
               y_ref, z2_ref, dinvm_ref):
    deg = degp0_ref[...] + degp1_ref[...] + 1.0
    di = lax.rsqrt(deg)
    dinvm_ref[...] = jnp.where(mark_ref[...] > 0.0, di, 0.0)
    x = jnp.dot(emb_ref[...], w_ref[...], preferred_element_type=jnp.float32)
    y = x * di[:N, None]
    y_ref[...] = y
    z2_ref[pl.ds(0, N), :] = 0.5 * di[:N, None] * y
    z2_ref[pl.ds(N, NPAD - N), :] = jnp.zeros((NPAD - N, C), jnp.float32)


_prep = pl.pallas_call(
    _prep_body,
    out_shape=[
        jax.ShapeDtypeStruct((N, C), jnp.float32),
        jax.ShapeDtypeStruct((NPAD, C), jnp.float32),
        jax.ShapeDtypeStruct((NPAD,), jnp.float32),
    ],
)


@functools.partial(
    pl.kernel,
    mesh=_mesh,
    out_type=[
        jax.ShapeDtypeStruct((NPAD, C), jnp.float32),
        jax.ShapeDtypeStruct((NPAD, C), jnp.float32),
    ],
    scratch_types=[
        pltpu.VMEM((SB, CHW), jnp.int32),
        pltpu.VMEM((SB, CHW), jnp.int32),
        pltpu.VMEM((SB, CHW), jnp.float32),
        pltpu.VMEM((NPAD,), jnp.float32),
        pltpu.VMEM((FCAP,), jnp.int32),
        pltpu.VMEM((FCAP,), jnp.int32),
        pltpu.VMEM((FCAP,), jnp.float32),
        pltpu.VMEM((PBW,), jnp.int32),
        pltpu.VMEM((PBW,), jnp.int32),
        pltpu.VMEM((PBW,), jnp.int32),
        pltpu.VMEM((PBW,), jnp.int32),
        pltpu.VMEM((PBW, C), jnp.float32),
        pltpu.VMEM((PBW, C), jnp.float32),
        pltpu.VMEM((PBW, C), jnp.float32),
        pltpu.VMEM((PBW, C), jnp.float32),
        pltpu.VMEM_SHARED((NPAD, C), jnp.float32),
        pltpu.SemaphoreType.DMA,
        pltpu.SemaphoreType.DMA,
        pltpu.SemaphoreType.DMA,
        pltpu.SemaphoreType.DMA,
    ],
    compiler_params=_sc_params,
)
def _scatter(row2d, col2d, ew2d, dinvm_hbm, y_hbm, z2_hbm, part0, part1,
             row_v, col_v, ew_v, dinvm_v, row_f, col_f, nrm_f, cb0, cb1,
             rb0, rb1, g0, g1, s0, s1, acc, semg0, semg1, sems0, sems1):
    c = lax.axis_index("c")
    s = lax.axis_index("s")
    wid = c * NSUB + s
    pltpu.sync_copy(dinvm_hbm, dinvm_v)
    pltpu.sync_copy(z2_hbm.at[pl.ds(s * RPS, RPS)], acc.at[pl.ds(s * RPS, RPS)])
    plsc.subcore_barrier()

    def blk(b, cnt):
        base = wid * NCH + b * SB
        pltpu.sync_copy(row2d.at[pl.ds(base, SB)], row_v)
        pltpu.sync_copy(col2d.at[pl.ds(base, SB)], col_v)
        pltpu.sync_copy(ew2d.at[pl.ds(base, SB)], ew_v)

        def compact(j, cnt2):
            for g in range(CHW // 16):
                c16 = col_v[j, pl.ds(g * 16, 16)]
                dm = plsc.load_gather(dinvm_v, [c16])
                m = dm > 0.0
                r16 = row_v[j, pl.ds(g * 16, 16)]
                w16 = ew_v[j, pl.ds(g * 16, 16)]
                plsc.store_compressed(row_f.at[pl.ds(cnt2, 16)], r16, mask=m)
                plsc.store_compressed(col_f.at[pl.ds(cnt2, 16)], c16, mask=m)
                plsc.store_compressed(nrm_f.at[pl.ds(cnt2, 16)], w16 * dm, mask=m)
                cnt2 = cnt2 + plsc.all_reduce_population_count(m)[0]
            return cnt2

        return lax.fori_loop(0, SB, compact, cnt)

    cnt = lax.fori_loop(0, NCH // SB, blk, jnp.int32(0))
    for k in range(2 * PBW // 16):
        row_f[pl.ds(cnt + k * 16, 16)] = jnp.zeros((16,), jnp.int32)
        col_f[pl.ds(cnt + k * 16, 16)] = jnp.zeros((16,), jnp.int32)
        nrm_f[pl.ds(cnt + k * 16, 16)] = jnp.zeros((16,), jnp.float32)
    nchf = (cnt + PBW - 1) // PBW
    npair = (nchf + 1) // 2

    def scale(base, gbuf, sbuf):
        def body(g, icarry):
            norm16 = nrm_f[pl.ds(base + g * 16, 16)]
            for l in range(16):
                e = g * 16 + l
                ns = norm16[l]
                for q in range(C // 16):
                    sbuf[e, pl.ds(q * 16, 16)] = gbuf[e, pl.ds(q * 16, 16)] * ns
            return icarry

        lax.fori_loop(0, PBW // 16, body, None)

    def cidx(base, ibuf, src_f):
        def body(g, icarry):
            ibuf[pl.ds(g * 16, 16)] = src_f[pl.ds(base + g * 16, 16)]
            return icarry

        lax.fori_loop(0, PBW // 16, body, None)

    @pl.when(npair > 0)
    def _():
        cidx(0, rb0, row_f)
        pltpu.async_copy(y_hbm.at[rb0], g0, semg0)

        def chunk_pair(t, carry):
            b0 = (2 * t) * PBW
            b1 = (2 * t + 1) * PBW
            cidx(b1, rb1, row_f)
            pltpu.make_async_copy(y_hbm.at[rb0], g0, semg0).wait()
            pltpu.async_copy(y_hbm.at[rb1], g1, semg1)
            cidx(b0, cb0, col_f)
            scale(b0, g0, s0)
            h0 = pltpu.async_copy(s0, acc.at[cb0], sems0, add=True)

            @pl.when(t < npair - 1)
            def _():
                cidx(b0 + 2 * PBW, rb0, row_f)
                pltpu.async_copy(y_hbm.at[rb0], g0, semg0)

            pltpu.make_async_copy(y_hbm.at[rb1], g1, semg1).wait()
            cidx(b1, cb1, col_f)
            scale(b1, g1, s1)
            h1 = pltpu.async_copy(s1, acc.at[cb1], sems1, add=True)
            h0.wait()
            h1.wait()
            return carry

        lax.fori_loop(0, npair, chunk_pair, None)

    plsc.subcore_barrier()

    @pl.when(c == 0)
    def _():
        pltpu.sync_copy(acc.at[pl.ds(s * RPS, RPS)], part0.at[pl.ds(s * RPS, RPS)])

    @pl.when(c == 1)
    def _():
        pltpu.sync_copy(acc.at[pl.ds(s * RPS, RPS)], part1.at[pl.ds(s * RPS, RPS)])


@functools.partial(
    pl.kernel,
    mesh=_mesh,
    out_type=jax.ShapeDtypeStruct((B, C), jnp.float32),
    scratch_types=[
        pltpu.VMEM((BW,), jnp.int32),
        pltpu.VMEM((BW, C), jnp.float32),
        pltpu.VMEM((BW, C), jnp.float32),
        pltpu.SemaphoreType.DMA,
    ],
    compiler_params=_sc_params,
)
def _combine(nodes1d, part0, part1, out, idx_v, buf0, buf1, sem):
    c = lax.axis_index("c")
    s = lax.axis_index("s")
    wid = c * NSUB + s
    pltpu.sync_copy(nodes1d.at[pl.ds(wid * BW, BW)], idx_v)
    pltpu.async_copy(part0.at[idx_v], buf0, sem).wait()
    pltpu.async_copy(part1.at[idx_v], buf1, sem).wait()

    def addrow(e, carry):
        for q in range(C // 16):
            buf0[e, pl.ds(q * 16, 16)] = (
                buf0[e, pl.ds(q * 16, 16)] + buf1[e, pl.ds(q * 16, 16)]
            )
        return carry

    lax.fori_loop(0, BW, addrow, None)
    pltpu.sync_copy(buf0, out.at[pl.ds(wid * BW, BW)])


def kernel(nodes, edge_index, edge_weight, emb, W):
    rowp = edge_index[0].reshape(NW * NCH, CHW)
    colp = edge_index[1].reshape(NW * NCH, CHW)
    ewp = edge_weight.reshape(NW * NCH, CHW)
    nodes2d = nodes.reshape(B // MW, MW)

    degp0, degp1, mark = _deg(colp, ewp, nodes2d)
    y, z2, dinvm = _prep(emb, W, degp0, degp1, mark)
    part0, part1 = _scatter(rowp, colp, ewp, dinvm, y, z2)
    return _combine(nodes, part0, part1)


# speedup vs baseline: 1.2955x; 1.2955x over previous
import functools

import jax
import jax.numpy as jnp
from jax import lax
from jax.experimental import pallas as pl
from jax.experimental.pallas import tpu as pltpu
from jax.experimental.pallas import tpu_sc as plsc

N = 10000
E = 320000
D = 128
C = 64
B = 4096

NCORE = 2
NSUB = 16
NW = NCORE * NSUB
CHW = 80
NCH = E // (NW * CHW)
MW = 128
RPS = 640
NPAD = NSUB * RPS
BW = B // NW
FCAP = NCH * CHW + 2 * CHW
SB = 25

_mesh = plsc.VectorSubcoreMesh(core_axis_name="c", subcore_axis_name="s")
_sc_params = pltpu.CompilerParams(
    use_tc_tiling_on_sc=False, needs_layout_passes=False)


@functools.partial(
    pl.kernel,
    mesh=_mesh,
    out_type=[
        jax.ShapeDtypeStruct((NPAD,), jnp.float32),
        jax.ShapeDtypeStruct((NPAD,), jnp.float32),
        jax.ShapeDtypeStruct((NPAD,), jnp.float32),
    ],
    scratch_types=[
        pltpu.VMEM((NCH, CHW), jnp.int32),
        pltpu.VMEM((NCH, CHW), jnp.float32),
        pltpu.VMEM((B // NSUB // MW, MW), jnp.int32),
        pltpu.VMEM((RPS,), jnp.float32),
        pltpu.VMEM((MW,), jnp.float32),
        pltpu.VMEM_SHARED((NPAD,), jnp.float32),
        pltpu.VMEM_SHARED((NPAD,), jnp.float32),
    ],
    compiler_params=_sc_params,
)
def _deg(col2d, ew2d, nodes2d, degp0, degp1, mark, col_v, ew_v, node_v,
         zero_v, ones_v, deg_s, mark_s):
    c = lax.axis_index("c")
    s = lax.axis_index("s")
    wid = c * NSUB + s
    nrows = B // NSUB // MW
    pltpu.sync_copy(col2d.at[pl.ds(wid * NCH, NCH)], col_v)
    pltpu.sync_copy(ew2d.at[pl.ds(wid * NCH, NCH)], ew_v)
    pltpu.sync_copy(nodes2d.at[pl.ds(s * nrows, nrows)], node_v)

    def zb(i, carry):
        zero_v[pl.ds(i * 16, 16)] = jnp.zeros((16,), jnp.float32)
        return carry

    lax.fori_loop(0, RPS // 16, zb, None)
    for i in range(MW // 16):
        ones_v[pl.ds(i * 16, 16)] = jnp.full((16,), 1.0, jnp.float32)
    pltpu.sync_copy(zero_v, deg_s.at[pl.ds(s * RPS, RPS)])
    pltpu.sync_copy(zero_v, mark_s.at[pl.ds(s * RPS, RPS)])
    plsc.subcore_barrier()

    def chunk(j, carry):
        pltpu.sync_copy(ew_v.at[j], deg_s.at[col_v.at[j]], add=True)
        return carry

    lax.fori_loop(0, NCH, chunk, None)
    for k in range(B // NSUB // MW):
        pltpu.sync_copy(ones_v, mark_s.at[node_v.at[k]], add=True)
    plsc.subcore_barrier()

    @pl.when(c == 0)
    def _():
        pltpu.sync_copy(deg_s.at[pl.ds(s * RPS, RPS)], degp0.at[pl.ds(s * RPS, RPS)])
        pltpu.sync_copy(mark_s.at[pl.ds(s * RPS, RPS)], mark.at[pl.ds(s * RPS, RPS)])

    @pl.when(c == 1)
    def _():
        pltpu.sync_copy(deg_s.at[pl.ds(s * RPS, RPS)], degp1.at[pl.ds(s * RPS, RPS)])


def _prep_body(emb_ref, w_ref, degp0_ref, degp1_ref, mark_ref,
               y_ref, z2_ref, dinvm_ref):
    deg = degp0_ref[...] + degp1_ref[...] + 1.0
    di = lax.rsqrt(deg)
    dinvm_ref[...] = jnp.where(mark_ref[...] > 0.0, di, 0.0)
    x = jnp.dot(emb_ref[...], w_ref[...], preferred_element_type=jnp.float32)
    y = x * di[:N, None]
    y_ref[...] = y
    z2_ref[pl.ds(0, N), :] = 0.5 * di[:N, None] * y
    z2_ref[pl.ds(N, NPAD - N), :] = jnp.zeros((NPAD - N, C), jnp.float32)


_prep = pl.pallas_call(
    _prep_body,
    out_shape=[
        jax.ShapeDtypeStruct((N, C), jnp.float32),
        jax.ShapeDtypeStruct((NPAD, C), jnp.float32),
        jax.ShapeDtypeStruct((NPAD,), jnp.float32),
    ],
)


@functools.partial(
    pl.kernel,
    mesh=_mesh,
    out_type=[
        jax.ShapeDtypeStruct((NPAD, C), jnp.float32),
        jax.ShapeDtypeStruct((NPAD, C), jnp.float32),
    ],
    scratch_types=[
        pltpu.VMEM((SB, CHW), jnp.int32),
        pltpu.VMEM((SB, CHW), jnp.int32),
        pltpu.VMEM((SB, CHW), jnp.float32),
        pltpu.VMEM((NPAD,), jnp.float32),
        pltpu.VMEM((FCAP,), jnp.int32),
        pltpu.VMEM((FCAP,), jnp.int32),
        pltpu.VMEM((FCAP,), jnp.float32),
        pltpu.VMEM((CHW,), jnp.int32),
        pltpu.VMEM((CHW,), jnp.int32),
        pltpu.VMEM((CHW,), jnp.int32),
        pltpu.VMEM((CHW,), jnp.int32),
        pltpu.VMEM((CHW, C), jnp.float32),
        pltpu.VMEM((CHW, C), jnp.float32),
        pltpu.VMEM((CHW, C), jnp.float32),
        pltpu.VMEM((CHW, C), jnp.float32),
        pltpu.VMEM_SHARED((NPAD, C), jnp.float32),
        pltpu.SemaphoreType.DMA,
        pltpu.SemaphoreType.DMA,
        pltpu.SemaphoreType.DMA,
        pltpu.SemaphoreType.DMA,
    ],
    compiler_params=_sc_params,
)
def _scatter(row2d, col2d, ew2d, dinvm_hbm, y_hbm, z2_hbm, part0, part1,
             row_v, col_v, ew_v, dinvm_v, row_f, col_f, nrm_f, cb0, cb1,
             rb0, rb1, g0, g1, s0, s1, acc, semg0, semg1, sems0, sems1):
    c = lax.axis_index("c")
    s = lax.axis_index("s")
    wid = c * NSUB + s
    pltpu.sync_copy(dinvm_hbm, dinvm_v)
    pltpu.sync_copy(z2_hbm.at[pl.ds(s * RPS, RPS)], acc.at[pl.ds(s * RPS, RPS)])
    plsc.subcore_barrier()

    def blk(b, cnt):
        base = wid * NCH + b * SB
        pltpu.sync_copy(row2d.at[pl.ds(base, SB)], row_v)
        pltpu.sync_copy(col2d.at[pl.ds(base, SB)], col_v)
        pltpu.sync_copy(ew2d.at[pl.ds(base, SB)], ew_v)

        def compact(j, cnt2):
            for g in range(CHW // 16):
                c16 = col_v[j, pl.ds(g * 16, 16)]
                dm = plsc.load_gather(dinvm_v, [c16])
                m = dm > 0.0
                r16 = row_v[j, pl.ds(g * 16, 16)]
                w16 = ew_v[j, pl.ds(g * 16, 16)]
                plsc.store_compressed(row_f.at[pl.ds(cnt2, 16)], r16, mask=m)
                plsc.store_compressed(col_f.at[pl.ds(cnt2, 16)], c16, mask=m)
                plsc.store_compressed(nrm_f.at[pl.ds(cnt2, 16)], w16 * dm, mask=m)
                cnt2 = cnt2 + plsc.all_reduce_population_count(m)[0]
            return cnt2

        return lax.fori_loop(0, SB, compact, cnt)

    cnt = lax.fori_loop(0, NCH // SB, blk, jnp.int32(0))
    for k in range(2 * CHW // 16):
        row_f[pl.ds(cnt + k * 16, 16)] = jnp.zeros((16,), jnp.int32)
        col_f[pl.ds(cnt + k * 16, 16)] = jnp.zeros((16,), jnp.int32)
        nrm_f[pl.ds(cnt + k * 16, 16)] = jnp.zeros((16,), jnp.float32)
    nchf = (cnt + CHW - 1) // CHW
    npair = (nchf + 1) // 2

    def scale(base, gbuf, sbuf):
        def body(g, icarry):
            norm16 = nrm_f[pl.ds(base + g * 16, 16)]
            for l in range(16):
                e = g * 16 + l
                ns = norm16[l]
                for q in range(C // 16):
                    sbuf[e, pl.ds(q * 16, 16)] = gbuf[e, pl.ds(q * 16, 16)] * ns
            return icarry

        lax.fori_loop(0, CHW // 16, body, None)

    def cidx(base, ibuf, src_f):
        def body(g, icarry):
            ibuf[pl.ds(g * 16, 16)] = src_f[pl.ds(base + g * 16, 16)]
            return icarry

        lax.fori_loop(0, CHW // 16, body, None)

    @pl.when(npair > 0)
    def _():
        cidx(0, rb0, row_f)
        pltpu.async_copy(y_hbm.at[rb0], g0, semg0)

        def chunk_pair(t, carry):
            b0 = (2 * t) * CHW
            b1 = (2 * t + 1) * CHW
            cidx(b1, rb1, row_f)
            pltpu.make_async_copy(y_hbm.at[rb0], g0, semg0).wait()
            pltpu.async_copy(y_hbm.at[rb1], g1, semg1)

            @pl.when(t > 0)
            def _():
                pltpu.make_async_copy(s0, acc.at[cb0], sems0).wait()
                pltpu.make_async_copy(s1, acc.at[cb1], sems1).wait()

            cidx(b0, cb0, col_f)
            scale(b0, g0, s0)
            pltpu.async_copy(s0, acc.at[cb0], sems0, add=True)

            @pl.when(t < npair - 1)
            def _():
                cidx(b0 + 2 * CHW, rb0, row_f)
                pltpu.async_copy(y_hbm.at[rb0], g0, semg0)

            pltpu.make_async_copy(y_hbm.at[rb1], g1, semg1).wait()
            cidx(b1, cb1, col_f)
            scale(b1, g1, s1)
            pltpu.async_copy(s1, acc.at[cb1], sems1, add=True)
            return carry

        lax.fori_loop(0, npair, chunk_pair, None)
        pltpu.make_async_copy(s0, acc.at[cb0], sems0).wait()
        pltpu.make_async_copy(s1, acc.at[cb1], sems1).wait()

    plsc.subcore_barrier()

    @pl.when(c == 0)
    def _():
        pltpu.sync_copy(acc.at[pl.ds(s * RPS, RPS)], part0.at[pl.ds(s * RPS, RPS)])

    @pl.when(c == 1)
    def _():
        pltpu.sync_copy(acc.at[pl.ds(s * RPS, RPS)], part1.at[pl.ds(s * RPS, RPS)])


@functools.partial(
    pl.kernel,
    mesh=_mesh,
    out_type=jax.ShapeDtypeStruct((B, C), jnp.float32),
    scratch_types=[
        pltpu.VMEM((BW,), jnp.int32),
        pltpu.VMEM((BW, C), jnp.float32),
        pltpu.VMEM((BW, C), jnp.float32),
        pltpu.SemaphoreType.DMA,
    ],
    compiler_params=_sc_params,
)
def _combine(nodes1d, part0, part1, out, idx_v, buf0, buf1, sem):
    c = lax.axis_index("c")
    s = lax.axis_index("s")
    wid = c * NSUB + s
    pltpu.sync_copy(nodes1d.at[pl.ds(wid * BW, BW)], idx_v)
    pltpu.async_copy(part0.at[idx_v], buf0, sem).wait()
    pltpu.async_copy(part1.at[idx_v], buf1, sem).wait()

    def addrow(e, carry):
        for q in range(C // 16):
            buf0[e, pl.ds(q * 16, 16)] = (
                buf0[e, pl.ds(q * 16, 16)] + buf1[e, pl.ds(q * 16, 16)]
            )
        return carry

    lax.fori_loop(0, BW, addrow, None)
    pltpu.sync_copy(buf0, out.at[pl.ds(wid * BW, BW)])


def kernel(nodes, edge_index, edge_weight, emb, W):
    rowp = edge_index[0].reshape(NW * NCH, CHW)
    colp = edge_index[1].reshape(NW * NCH, CHW)
    ewp = edge_weight.reshape(NW * NCH, CHW)
    nodes2d = nodes.reshape(B // MW, MW)

    degp0, degp1, mark = _deg(colp, ewp, nodes2d)
    y, z2, dinvm = _prep(emb, W, degp0, degp1, mark)
    part0, part1 = _scatter(rowp, colp, ewp, dinvm, y, z2)
    return _combine(nodes, part0, part1)


# speedup vs baseline: 1.3479x; 1.0404x over previous
import functools

import jax
import jax.numpy as jnp
from jax import lax
from jax.experimental import pallas as pl
from jax.experimental.pallas import tpu as pltpu
from jax.experimental.pallas import tpu_sc as plsc

N = 10000
E = 320000
D = 128
C = 64
B = 4096

NCORE = 2
NSUB = 16
NW = NCORE * NSUB
CHW = 80
NCH = E // (NW * CHW)
MW = 128
RPS = 640
NPAD = NSUB * RPS
BW = B // NW
FCAP = NCH * CHW + 2 * CHW
SB = 25

_mesh = plsc.VectorSubcoreMesh(core_axis_name="c", subcore_axis_name="s")
_sc_params = pltpu.CompilerParams(
    use_tc_tiling_on_sc=False, needs_layout_passes=False)


@functools.partial(
    pl.kernel,
    mesh=_mesh,
    out_type=[
        jax.ShapeDtypeStruct((NPAD,), jnp.float32),
        jax.ShapeDtypeStruct((NPAD,), jnp.float32),
        jax.ShapeDtypeStruct((NPAD,), jnp.float32),
    ],
    scratch_types=[
        pltpu.VMEM((NCH, CHW), jnp.int32),
        pltpu.VMEM((NCH, CHW), jnp.float32),
        pltpu.VMEM((B // NSUB // MW, MW), jnp.int32),
        pltpu.VMEM((RPS,), jnp.float32),
        pltpu.VMEM((MW,), jnp.float32),
        pltpu.VMEM_SHARED((NPAD,), jnp.float32),
        pltpu.VMEM_SHARED((NPAD,), jnp.float32),
        pltpu.SemaphoreType.DMA,
    ],
    compiler_params=_sc_params,
)
def _deg(col2d, ew2d, nodes2d, degp0, degp1, mark, col_v, ew_v, node_v,
         zero_v, ones_v, deg_s, mark_s, semd):
    c = lax.axis_index("c")
    s = lax.axis_index("s")
    wid = c * NSUB + s
    nrows = B // NSUB // MW
    pltpu.sync_copy(col2d.at[pl.ds(wid * NCH, NCH)], col_v)
    pltpu.sync_copy(ew2d.at[pl.ds(wid * NCH, NCH)], ew_v)
    pltpu.sync_copy(nodes2d.at[pl.ds(s * nrows, nrows)], node_v)

    def zb(i, carry):
        zero_v[pl.ds(i * 16, 16)] = jnp.zeros((16,), jnp.float32)
        return carry

    lax.fori_loop(0, RPS // 16, zb, None)
    for i in range(MW // 16):
        ones_v[pl.ds(i * 16, 16)] = jnp.full((16,), 1.0, jnp.float32)
    pltpu.sync_copy(zero_v, deg_s.at[pl.ds(s * RPS, RPS)])
    pltpu.sync_copy(zero_v, mark_s.at[pl.ds(s * RPS, RPS)])
    plsc.subcore_barrier()

    K8 = 8

    def blk8(b, carry):
        for k in range(K8):
            pltpu.async_copy(ew_v.at[b * K8 + k], deg_s.at[col_v.at[b * K8 + k]],
                             semd, add=True)
        for k in range(K8):
            pltpu.make_async_copy(ew_v.at[b * K8 + k], deg_s.at[col_v.at[b * K8 + k]],
                                  semd).wait()
        return carry

    lax.fori_loop(0, NCH // K8, blk8, None)
    for j in range(NCH - NCH % K8, NCH):
        pltpu.async_copy(ew_v.at[j], deg_s.at[col_v.at[j]], semd, add=True)
    for j in range(NCH - NCH % K8, NCH):
        pltpu.make_async_copy(ew_v.at[j], deg_s.at[col_v.at[j]], semd).wait()
    for k in range(B // NSUB // MW):
        pltpu.sync_copy(ones_v, mark_s.at[node_v.at[k]], add=True)
    plsc.subcore_barrier()

    @pl.when(c == 0)
    def _():
        pltpu.sync_copy(deg_s.at[pl.ds(s * RPS, RPS)], degp0.at[pl.ds(s * RPS, RPS)])
        pltpu.sync_copy(mark_s.at[pl.ds(s * RPS, RPS)], mark.at[pl.ds(s * RPS, RPS)])

    @pl.when(c == 1)
    def _():
        pltpu.sync_copy(deg_s.at[pl.ds(s * RPS, RPS)], degp1.at[pl.ds(s * RPS, RPS)])


def _prep_body(emb_ref, w_ref, degp0_ref, degp1_ref, mark_ref,
               y_ref, z2_ref, dinvm_ref):
    deg = degp0_ref[...] + degp1_ref[...] + 1.0
    di = lax.rsqrt(deg)
    dinvm_ref[...] = jnp.where(mark_ref[...] > 0.0, di, 0.0)
    x = jnp.dot(emb_ref[...], w_ref[...], preferred_element_type=jnp.float32)
    y = x * di[:N, None]
    y_ref[...] = y
    z2_ref[pl.ds(0, N), :] = 0.5 * di[:N, None] * y
    z2_ref[pl.ds(N, NPAD - N), :] = jnp.zeros((NPAD - N, C), jnp.float32)


_prep = pl.pallas_call(
    _prep_body,
    out_shape=[
        jax.ShapeDtypeStruct((N, C), jnp.float32),
        jax.ShapeDtypeStruct((NPAD, C), jnp.float32),
        jax.ShapeDtypeStruct((NPAD,), jnp.float32),
    ],
)


@functools.partial(
    pl.kernel,
    mesh=_mesh,
    out_type=[
        jax.ShapeDtypeStruct((NPAD, C), jnp.float32),
        jax.ShapeDtypeStruct((NPAD, C), jnp.float32),
    ],
    scratch_types=[
        pltpu.VMEM((SB, CHW), jnp.int32),
        pltpu.VMEM((SB, CHW), jnp.int32),
        pltpu.VMEM((SB, CHW), jnp.float32),
        pltpu.VMEM((NPAD,), jnp.float32),
        pltpu.VMEM((FCAP,), jnp.int32),
        pltpu.VMEM((FCAP,), jnp.int32),
        pltpu.VMEM((FCAP,), jnp.float32),
        pltpu.VMEM((CHW,), jnp.int32),
        pltpu.VMEM((CHW,), jnp.int32),
        pltpu.VMEM((CHW,), jnp.int32),
        pltpu.VMEM((CHW,), jnp.int32),
        pltpu.VMEM((CHW, C), jnp.float32),
        pltpu.VMEM((CHW, C), jnp.float32),
        pltpu.VMEM((CHW, C), jnp.float32),
        pltpu.VMEM((CHW, C), jnp.float32),
        pltpu.VMEM_SHARED((NPAD, C), jnp.float32),
        pltpu.SemaphoreType.DMA,
        pltpu.SemaphoreType.DMA,
        pltpu.SemaphoreType.DMA,
        pltpu.SemaphoreType.DMA,
    ],
    compiler_params=_sc_params,
)
def _scatter(row2d, col2d, ew2d, dinvm_hbm, y_hbm, z2_hbm, part0, part1,
             row_v, col_v, ew_v, dinvm_v, row_f, col_f, nrm_f, cb0, cb1,
             rb0, rb1, g0, g1, s0, s1, acc, semg0, semg1, sems0, sems1):
    c = lax.axis_index("c")
    s = lax.axis_index("s")
    wid = c * NSUB + s
    pltpu.sync_copy(dinvm_hbm, dinvm_v)
    pltpu.sync_copy(z2_hbm.at[pl.ds(s * RPS, RPS)], acc.at[pl.ds(s * RPS, RPS)])
    plsc.subcore_barrier()

    def blk(b, cnt):
        base = wid * NCH + b * SB
        pltpu.sync_copy(row2d.at[pl.ds(base, SB)], row_v)
        pltpu.sync_copy(col2d.at[pl.ds(base, SB)], col_v)
        pltpu.sync_copy(ew2d.at[pl.ds(base, SB)], ew_v)

        def compact(j, cnt2):
            for g in range(CHW // 16):
                c16 = col_v[j, pl.ds(g * 16, 16)]
                dm = plsc.load_gather(dinvm_v, [c16])
                m = dm > 0.0
                r16 = row_v[j, pl.ds(g * 16, 16)]
                w16 = ew_v[j, pl.ds(g * 16, 16)]
                plsc.store_compressed(row_f.at[pl.ds(cnt2, 16)], r16, mask=m)
                plsc.store_compressed(col_f.at[pl.ds(cnt2, 16)], c16, mask=m)
                plsc.store_compressed(nrm_f.at[pl.ds(cnt2, 16)], w16 * dm, mask=m)
                cnt2 = cnt2 + plsc.all_reduce_population_count(m)[0]
            return cnt2

        return lax.fori_loop(0, SB, compact, cnt)

    cnt = lax.fori_loop(0, NCH // SB, blk, jnp.int32(0))
    for k in range(2 * CHW // 16):
        row_f[pl.ds(cnt + k * 16, 16)] = jnp.zeros((16,), jnp.int32)
        col_f[pl.ds(cnt + k * 16, 16)] = jnp.zeros((16,), jnp.int32)
        nrm_f[pl.ds(cnt + k * 16, 16)] = jnp.zeros((16,), jnp.float32)
    nchf = (cnt + CHW - 1) // CHW
    npair = (nchf + 1) // 2

    def scale(base, gbuf, sbuf):
        def body(g, icarry):
            norm16 = nrm_f[pl.ds(base + g * 16, 16)]
            for l in range(16):
                e = g * 16 + l
                ns = norm16[l]
                for q in range(C // 16):
                    sbuf[e, pl.ds(q * 16, 16)] = gbuf[e, pl.ds(q * 16, 16)] * ns
            return icarry

        lax.fori_loop(0, CHW // 16, body, None)

    def cidx(base, ibuf, src_f):
        def body(g, icarry):
            ibuf[pl.ds(g * 16, 16)] = src_f[pl.ds(base + g * 16, 16)]
            return icarry

        lax.fori_loop(0, CHW // 16, body, None)

    @pl.when(npair > 0)
    def _():
        cidx(0, rb0, row_f)
        pltpu.async_copy(y_hbm.at[rb0], g0, semg0)

        def chunk_pair(t, carry):
            b0 = (2 * t) * CHW
            b1 = (2 * t + 1) * CHW
            cidx(b1, rb1, row_f)
            pltpu.make_async_copy(y_hbm.at[rb0], g0, semg0).wait()
            pltpu.async_copy(y_hbm.at[rb1], g1, semg1)

            @pl.when(t > 0)
            def _():
                pltpu.make_async_copy(s0, acc.at[cb0], sems0).wait()
                pltpu.make_async_copy(s1, acc.at[cb1], sems1).wait()

            cidx(b0, cb0, col_f)
            scale(b0, g0, s0)
            pltpu.async_copy(s0, acc.at[cb0], sems0, add=True)

            @pl.when(t < npair - 1)
            def _():
                cidx(b0 + 2 * CHW, rb0, row_f)
                pltpu.async_copy(y_hbm.at[rb0], g0, semg0)

            pltpu.make_async_copy(y_hbm.at[rb1], g1, semg1).wait()
            cidx(b1, cb1, col_f)
            scale(b1, g1, s1)
            pltpu.async_copy(s1, acc.at[cb1], sems1, add=True)
            return carry

        lax.fori_loop(0, npair, chunk_pair, None)
        pltpu.make_async_copy(s0, acc.at[cb0], sems0).wait()
        pltpu.make_async_copy(s1, acc.at[cb1], sems1).wait()

    plsc.subcore_barrier()

    @pl.when(c == 0)
    def _():
        pltpu.sync_copy(acc.at[pl.ds(s * RPS, RPS)], part0.at[pl.ds(s * RPS, RPS)])

    @pl.when(c == 1)
    def _():
        pltpu.sync_copy(acc.at[pl.ds(s * RPS, RPS)], part1.at[pl.ds(s * RPS, RPS)])


@functools.partial(
    pl.kernel,
    mesh=_mesh,
    out_type=jax.ShapeDtypeStruct((B, C), jnp.float32),
    scratch_types=[
        pltpu.VMEM((BW,), jnp.int32),
        pltpu.VMEM((BW, C), jnp.float32),
        pltpu.VMEM((BW, C), jnp.float32),
        pltpu.SemaphoreType.DMA,
    ],
    compiler_params=_sc_params,
)
def _combine(nodes1d, part0, part1, out, idx_v, buf0, buf1, sem):
    c = lax.axis_index("c")
    s = lax.axis_index("s")
    wid = c * NSUB + s
    pltpu.sync_copy(nodes1d.at[pl.ds(wid * BW, BW)], idx_v)
    pltpu.async_copy(part0.at[idx_v], buf0, sem).wait()
    pltpu.async_copy(part1.at[idx_v], buf1, sem).wait()

    def addrow(e, carry):
        for q in range(C // 16):
            buf0[e, pl.ds(q * 16, 16)] = (
                buf0[e, pl.ds(q * 16, 16)] + buf1[e, pl.ds(q * 16, 16)]
            )
        return carry

    lax.fori_loop(0, BW, addrow, None)
    pltpu.sync_copy(buf0, out.at[pl.ds(wid * BW, BW)])


def kernel(nodes, edge_index, edge_weight, emb, W):
    rowp = edge_index[0].reshape(NW * NCH, CHW)
    colp = edge_index[1].reshape(NW * NCH, CHW)
    ewp = edge_weight.reshape(NW * NCH, CHW)
    nodes2d = nodes.reshape(B // MW, MW)

    degp0, degp1, mark = _deg(colp, ewp, nodes2d)
    y, z2, dinvm = _prep(emb, W, degp0, degp1, mark)
    part0, part1 = _scatter(rowp, colp, ewp, dinvm, y, z2)
    return _combine(nodes, part0, part1)


# speedup vs baseline: 1.4082x; 1.0447x over previous
import functools

import jax
import jax.numpy as jnp
from jax import lax
from jax.experimental import pallas as pl
from jax.experimental.pallas import tpu as pltpu
from jax.experimental.pallas import tpu_sc as plsc

N = 10000
E = 320000
D = 128
C = 64
B = 4096

NCORE = 2
NSUB = 16
NW = NCORE * NSUB
CHW = 80
NCH = E // (NW * CHW)
MW = 128
RPS = 640
NPAD = NSUB * RPS
BW = B // NW
FCAP = NCH * CHW + 2 * CHW
SB = 25

_mesh = plsc.VectorSubcoreMesh(core_axis_name="c", subcore_axis_name="s")
_sc_params = pltpu.CompilerParams(
    use_tc_tiling_on_sc=False, needs_layout_passes=False)


@functools.partial(
    pl.kernel,
    mesh=_mesh,
    out_type=[
        jax.ShapeDtypeStruct((NPAD,), jnp.float32),
        jax.ShapeDtypeStruct((NPAD,), jnp.float32),
        jax.ShapeDtypeStruct((NPAD,), jnp.float32),
    ],
    scratch_types=[
        pltpu.VMEM((NCH, CHW), jnp.int32),
        pltpu.VMEM((NCH, CHW), jnp.float32),
        pltpu.VMEM((B // NSUB // MW, MW), jnp.int32),
        pltpu.VMEM((RPS,), jnp.float32),
        pltpu.VMEM((MW,), jnp.float32),
        pltpu.VMEM_SHARED((NPAD,), jnp.float32),
        pltpu.VMEM_SHARED((NPAD,), jnp.float32),
        pltpu.SemaphoreType.DMA,
    ],
    compiler_params=_sc_params,
)
def _deg(col2d, ew2d, nodes2d, degp0, degp1, mark, col_v, ew_v, node_v,
         zero_v, ones_v, deg_s, mark_s, semd):
    c = lax.axis_index("c")
    s = lax.axis_index("s")
    wid = c * NSUB + s
    nrows = B // NSUB // MW
    pltpu.sync_copy(col2d.at[pl.ds(wid * NCH, NCH)], col_v)
    pltpu.sync_copy(ew2d.at[pl.ds(wid * NCH, NCH)], ew_v)
    pltpu.sync_copy(nodes2d.at[pl.ds(s * nrows, nrows)], node_v)

    def zb(i, carry):
        zero_v[pl.ds(i * 16, 16)] = jnp.zeros((16,), jnp.float32)
        return carry

    lax.fori_loop(0, RPS // 16, zb, None)
    for i in range(MW // 16):
        ones_v[pl.ds(i * 16, 16)] = jnp.full((16,), 1.0, jnp.float32)
    pltpu.sync_copy(zero_v, deg_s.at[pl.ds(s * RPS, RPS)])
    pltpu.sync_copy(zero_v, mark_s.at[pl.ds(s * RPS, RPS)])
    plsc.subcore_barrier()

    K8 = 8

    def blk8(b, carry):
        for k in range(K8):
            pltpu.async_copy(ew_v.at[b * K8 + k], deg_s.at[col_v.at[b * K8 + k]],
                             semd, add=True)
        for k in range(K8):
            pltpu.make_async_copy(ew_v.at[b * K8 + k], deg_s.at[col_v.at[b * K8 + k]],
                                  semd).wait()
        return carry

    lax.fori_loop(0, NCH // K8, blk8, None)
    for j in range(NCH - NCH % K8, NCH):
        pltpu.async_copy(ew_v.at[j], deg_s.at[col_v.at[j]], semd, add=True)
    for j in range(NCH - NCH % K8, NCH):
        pltpu.make_async_copy(ew_v.at[j], deg_s.at[col_v.at[j]], semd).wait()
    for k in range(B // NSUB // MW):
        pltpu.sync_copy(ones_v, mark_s.at[node_v.at[k]], add=True)
    plsc.subcore_barrier()

    @pl.when(c == 0)
    def _():
        pltpu.sync_copy(deg_s.at[pl.ds(s * RPS, RPS)], degp0.at[pl.ds(s * RPS, RPS)])
        pltpu.sync_copy(mark_s.at[pl.ds(s * RPS, RPS)], mark.at[pl.ds(s * RPS, RPS)])

    @pl.when(c == 1)
    def _():
        pltpu.sync_copy(deg_s.at[pl.ds(s * RPS, RPS)], degp1.at[pl.ds(s * RPS, RPS)])


def _prep_body(emb_ref, w_ref, degp0_ref, degp1_ref, mark_ref,
               y_ref, z2_ref, dinvm_ref):
    deg = degp0_ref[...] + degp1_ref[...] + 1.0
    di = lax.rsqrt(deg)
    dinvm_ref[...] = jnp.where(mark_ref[...] > 0.0, di, 0.0)
    x = jnp.dot(emb_ref[...], w_ref[...], preferred_element_type=jnp.float32)
    y = x * di[:N, None]
    y_ref[...] = y
    z2_ref[pl.ds(0, N), :] = 0.5 * di[:N, None] * y
    z2_ref[pl.ds(N, NPAD - N), :] = jnp.zeros((NPAD - N, C), jnp.float32)


_prep = pl.pallas_call(
    _prep_body,
    out_shape=[
        jax.ShapeDtypeStruct((N, C), jnp.float32),
        jax.ShapeDtypeStruct((NPAD, C), jnp.float32),
        jax.ShapeDtypeStruct((NPAD,), jnp.float32),
    ],
)


@functools.partial(
    pl.kernel,
    mesh=_mesh,
    out_type=[
        jax.ShapeDtypeStruct((NPAD, C), jnp.float32),
        jax.ShapeDtypeStruct((NPAD, C), jnp.float32),
    ],
    scratch_types=[
        pltpu.VMEM((SB, CHW), jnp.int32),
        pltpu.VMEM((SB, CHW), jnp.int32),
        pltpu.VMEM((SB, CHW), jnp.float32),
        pltpu.VMEM((SB, CHW), jnp.int32),
        pltpu.VMEM((SB, CHW), jnp.int32),
        pltpu.VMEM((SB, CHW), jnp.float32),
        pltpu.VMEM((NPAD,), jnp.float32),
        pltpu.VMEM((FCAP,), jnp.int32),
        pltpu.VMEM((FCAP,), jnp.int32),
        pltpu.VMEM((FCAP,), jnp.float32),
        pltpu.VMEM((CHW,), jnp.int32),
        pltpu.VMEM((CHW,), jnp.int32),
        pltpu.VMEM((CHW,), jnp.int32),
        pltpu.VMEM((CHW,), jnp.int32),
        pltpu.VMEM((CHW, C), jnp.float32),
        pltpu.VMEM((CHW, C), jnp.float32),
        pltpu.VMEM((CHW, C), jnp.float32),
        pltpu.VMEM((CHW, C), jnp.float32),
        pltpu.VMEM_SHARED((NPAD, C), jnp.float32),
        pltpu.SemaphoreType.DMA,
        pltpu.SemaphoreType.DMA,
        pltpu.SemaphoreType.DMA,
        pltpu.SemaphoreType.DMA,
        pltpu.SemaphoreType.DMA,
    ],
    compiler_params=_sc_params,
)
def _scatter(row2d, col2d, ew2d, dinvm_hbm, y_hbm, z2_hbm, part0, part1,
             row_a, col_a, ew_a, row_b, col_b, ew_b, dinvm_v, row_f, col_f,
             nrm_f, cb0, cb1, rb0, rb1, g0, g1, s0, s1, acc,
             semg0, semg1, sems0, sems1, sems):
    c = lax.axis_index("c")
    s = lax.axis_index("s")
    wid = c * NSUB + s

    bufs = [(row_a, col_a, ew_a), (row_b, col_b, ew_b)]

    def fire(b, rv, cv, wv):
        base = wid * NCH + b * SB
        pltpu.async_copy(row2d.at[pl.ds(base, SB)], rv, sems)
        pltpu.async_copy(col2d.at[pl.ds(base, SB)], cv, sems)
        pltpu.async_copy(ew2d.at[pl.ds(base, SB)], wv, sems)

    def drain(b, rv, cv, wv):
        base = wid * NCH + b * SB
        pltpu.make_async_copy(row2d.at[pl.ds(base, SB)], rv, sems).wait()
        pltpu.make_async_copy(col2d.at[pl.ds(base, SB)], cv, sems).wait()
        pltpu.make_async_copy(ew2d.at[pl.ds(base, SB)], wv, sems).wait()

    fire(0, *bufs[0])
    pltpu.sync_copy(dinvm_hbm, dinvm_v)
    pltpu.sync_copy(z2_hbm.at[pl.ds(s * RPS, RPS)], acc.at[pl.ds(s * RPS, RPS)])
    plsc.subcore_barrier()

    cnt = jnp.int32(0)
    NB = NCH // SB
    for b in range(NB):
        rv, cv, wv = bufs[b % 2]
        drain(b, rv, cv, wv)
        if b + 1 < NB:
            fire(b + 1, *bufs[(b + 1) % 2])

        def compact(j, cnt2, rv=rv, cv=cv, wv=wv):
            for g in range(CHW // 16):
                c16 = cv[j, pl.ds(g * 16, 16)]
                dm = plsc.load_gather(dinvm_v, [c16])
                m = dm > 0.0
                r16 = rv[j, pl.ds(g * 16, 16)]
                w16 = wv[j, pl.ds(g * 16, 16)]
                plsc.store_compressed(row_f.at[pl.ds(cnt2, 16)], r16, mask=m)
                plsc.store_compressed(col_f.at[pl.ds(cnt2, 16)], c16, mask=m)
                plsc.store_compressed(nrm_f.at[pl.ds(cnt2, 16)], w16 * dm, mask=m)
                cnt2 = cnt2 + plsc.all_reduce_population_count(m)[0]
            return cnt2

        cnt = lax.fori_loop(0, SB, compact, cnt)
    for k in range(2 * CHW // 16):
        row_f[pl.ds(cnt + k * 16, 16)] = jnp.zeros((16,), jnp.int32)
        col_f[pl.ds(cnt + k * 16, 16)] = jnp.zeros((16,), jnp.int32)
        nrm_f[pl.ds(cnt + k * 16, 16)] = jnp.zeros((16,), jnp.float32)
    nchf = (cnt + CHW - 1) // CHW
    npair = (nchf + 1) // 2

    def scale(base, gbuf, sbuf):
        def body(g, icarry):
            norm16 = nrm_f[pl.ds(base + g * 16, 16)]
            for l in range(16):
                e = g * 16 + l
                ns = norm16[l]
                for q in range(C // 16):
                    sbuf[e, pl.ds(q * 16, 16)] = gbuf[e, pl.ds(q * 16, 16)] * ns
            return icarry

        lax.fori_loop(0, CHW // 16, body, None)

    def cidx(base, ibuf, src_f):
        def body(g, icarry):
            ibuf[pl.ds(g * 16, 16)] = src_f[pl.ds(base + g * 16, 16)]
            return icarry

        lax.fori_loop(0, CHW // 16, body, None)

    @pl.when(npair > 0)
    def _():
        cidx(0, rb0, row_f)
        pltpu.async_copy(y_hbm.at[rb0], g0, semg0)

        def chunk_pair(t, carry):
            b0 = (2 * t) * CHW
            b1 = (2 * t + 1) * CHW
            cidx(b1, rb1, row_f)
            pltpu.make_async_copy(y_hbm.at[rb0], g0, semg0).wait()
            pltpu.async_copy(y_hbm.at[rb1], g1, semg1)

            @pl.when(t > 0)
            def _():
                pltpu.make_async_copy(s0, acc.at[cb0], sems0).wait()
                pltpu.make_async_copy(s1, acc.at[cb1], sems1).wait()

            cidx(b0, cb0, col_f)
            scale(b0, g0, s0)
            pltpu.async_copy(s0, acc.at[cb0], sems0, add=True)

            @pl.when(t < npair - 1)
            def _():
                cidx(b0 + 2 * CHW, rb0, row_f)
                pltpu.async_copy(y_hbm.at[rb0], g0, semg0)

            pltpu.make_async_copy(y_hbm.at[rb1], g1, semg1).wait()
            cidx(b1, cb1, col_f)
            scale(b1, g1, s1)
            pltpu.async_copy(s1, acc.at[cb1], sems1, add=True)
            return carry

        lax.fori_loop(0, npair, chunk_pair, None)
        pltpu.make_async_copy(s0, acc.at[cb0], sems0).wait()
        pltpu.make_async_copy(s1, acc.at[cb1], sems1).wait()

    plsc.subcore_barrier()

    @pl.when(c == 0)
    def _():
        pltpu.sync_copy(acc.at[pl.ds(s * RPS, RPS)], part0.at[pl.ds(s * RPS, RPS)])

    @pl.when(c == 1)
    def _():
        pltpu.sync_copy(acc.at[pl.ds(s * RPS, RPS)], part1.at[pl.ds(s * RPS, RPS)])


@functools.partial(
    pl.kernel,
    mesh=_mesh,
    out_type=jax.ShapeDtypeStruct((B, C), jnp.float32),
    scratch_types=[
        pltpu.VMEM((BW,), jnp.int32),
        pltpu.VMEM((BW, C), jnp.float32),
        pltpu.VMEM((BW, C), jnp.float32),
        pltpu.SemaphoreType.DMA,
    ],
    compiler_params=_sc_params,
)
def _combine(nodes1d, part0, part1, out, idx_v, buf0, buf1, sem):
    c = lax.axis_index("c")
    s = lax.axis_index("s")
    wid = c * NSUB + s
    pltpu.sync_copy(nodes1d.at[pl.ds(wid * BW, BW)], idx_v)
    pltpu.async_copy(part0.at[idx_v], buf0, sem).wait()
    pltpu.async_copy(part1.at[idx_v], buf1, sem).wait()

    def addrow(e, carry):
        for q in range(C // 16):
            buf0[e, pl.ds(q * 16, 16)] = (
                buf0[e, pl.ds(q * 16, 16)] + buf1[e, pl.ds(q * 16, 16)]
            )
        return carry

    lax.fori_loop(0, BW, addrow, None)
    pltpu.sync_copy(buf0, out.at[pl.ds(wid * BW, BW)])


def kernel(nodes, edge_index, edge_weight, emb, W):
    rowp = edge_index[0].reshape(NW * NCH, CHW)
    colp = edge_index[1].reshape(NW * NCH, CHW)
    ewp = edge_weight.reshape(NW * NCH, CHW)
    nodes2d = nodes.reshape(B // MW, MW)

    degp0, degp1, mark = _deg(colp, ewp, nodes2d)
    y, z2, dinvm = _prep(emb, W, degp0, degp1, mark)
    part0, part1 = _scatter(rowp, colp, ewp, dinvm, y, z2)
    return _combine(nodes, part0, part1)
